# baseline (speedup 1.0000x reference)
import functools

import jax
import jax.numpy as jnp
from jax.experimental import pallas as pl
from jax.experimental.pallas import tpu as pltpu

_BK = 512
_NV = 4
_NA = 4


def _ln_relu(x, g, b, eps=1e-5):
    mu = jnp.mean(x, axis=-1, keepdims=True)
    var = jnp.mean((x - mu) ** 2, axis=-1, keepdims=True)
    y = (x - mu) * jax.lax.rsqrt(var + eps) * g + b
    return jnp.maximum(y, 0.0)


_DN_T = (((1,), (1,)), ((), ()))


def _gin_kernel(*refs, nb, n):
    v_refs = refs[:_NV]
    a_refs = refs[_NV:_NV + _NA]
    (eps_ref, w1_ref, b1_ref, g1_ref, be1_ref, w2_ref, b2_ref, g2_ref,
     be2_ref, o_ref, u_ref, h_ref, w1bf_ref) = refs[_NV + _NA:]
    k = pl.program_id(0)
    bf = jnp.bfloat16
    vb = _BK // _NV
    ab = n // _NA

    @pl.when(k == 0)
    def _cast_w1():
        w1bf_ref[...] = w1_ref[...].astype(bf)

    us = []
    for s in range(_NV):
        u_s = jax.lax.dot_general(
            v_refs[s][...].astype(bf), w1bf_ref[...], _DN_T,
            preferred_element_type=jnp.float32).astype(bf)
        u_ref[pl.ds(k * _BK + s * vb, vb), :] = u_s
        us.append(u_s)
    u_k = jnp.concatenate(us, axis=0)

    parts = [jnp.dot(a_refs[s][...].astype(bf), u_k,
                     preferred_element_type=jnp.float32)
             for s in range(_NA)]

    @pl.when(k == 0)
    def _init():
        for s in range(_NA):
            h_ref[pl.ds(s * ab, ab), :] = parts[s]

    @pl.when(k > 0)
    def _acc():
        for s in range(_NA):
            h_ref[pl.ds(s * ab, ab), :] += parts[s]

    @pl.when(k == nb - 1)
    def _epilogue():
        h = h_ref[...] + eps_ref[0, 0] * u_ref[...].astype(jnp.float32)
        h = h + b1_ref[...]
        h = _ln_relu(h, g1_ref[...], be1_ref[...])
        h2 = jax.lax.dot_general(h.astype(bf), w2_ref[...].astype(bf), _DN_T,
                                 preferred_element_type=jnp.float32)
        h2 = h2 + b2_ref[...]
        o_ref[...] = _ln_relu(h2, g2_ref[...], be2_ref[...])


def kernel(v, a, epsilon, W1, b1, g1, be1, W2, b2, g2, be2):
    n, _ = a.shape
    hid = W1.shape[0]
    out_dim = W2.shape[0]
    nb = n // _BK
    vb = _BK // _NV
    ab = n // _NA

    row = lambda x: x.reshape(1, -1)
    const = lambda k: (0, 0)
    v_specs = [pl.BlockSpec((vb, n), functools.partial(
        lambda k, s: (_NV * k + s, 0), s=s)) for s in range(_NV)]
    a_specs = [pl.BlockSpec((ab, _BK), functools.partial(
        lambda k, s: (s, k), s=s)) for s in range(_NA)]
    out = pl.pallas_call(
        functools.partial(_gin_kernel, nb=nb, n=n),
        grid=(nb,),
        in_specs=v_specs + a_specs + [
            pl.BlockSpec((1, 1), const),
            pl.BlockSpec((hid, n), const),
            pl.BlockSpec((1, hid), const),
            pl.BlockSpec((1, hid), const),
            pl.BlockSpec((1, hid), const),
            pl.BlockSpec((out_dim, hid), const),
            pl.BlockSpec((1, out_dim), const),
            pl.BlockSpec((1, out_dim), const),
            pl.BlockSpec((1, out_dim), const),
        ],
        out_specs=pl.BlockSpec((n, out_dim), const),
        out_shape=jax.ShapeDtypeStruct((n, out_dim), jnp.float32),
        scratch_shapes=[pltpu.VMEM((n, hid), jnp.bfloat16),
                        pltpu.VMEM((n, hid), jnp.float32),
                        pltpu.VMEM((hid, n), jnp.bfloat16)],
    )(*([v] * _NV), *([a] * _NA), epsilon, W1, row(b1), row(g1), row(be1),
      W2, row(b2), row(g2), row(be2))
    return out


# speedup vs baseline: 1.0934x; 1.0934x over previous
import functools

import jax
import jax.numpy as jnp
from jax.experimental import pallas as pl
from jax.experimental.pallas import tpu as pltpu

_BK = 512
_NV = 2
_NA = 2


def _ln_relu(x, g, b, eps=1e-5):
    mu = jnp.mean(x, axis=-1, keepdims=True)
    var = jnp.mean((x - mu) ** 2, axis=-1, keepdims=True)
    y = (x - mu) * jax.lax.rsqrt(var + eps) * g + b
    return jnp.maximum(y, 0.0)


_DN_T = (((1,), (1,)), ((), ()))


def _gin_kernel(*refs, nb, n):
    v_refs = refs[:_NV]
    a_refs = refs[_NV:_NV + _NA]
    (eps_ref, w1_ref, b1_ref, g1_ref, be1_ref, w2_ref, b2_ref, g2_ref,
     be2_ref, o_ref, u_ref, h_ref, w1bf_ref) = refs[_NV + _NA:]
    k = pl.program_id(0)
    bf = jnp.bfloat16
    vb = _BK // _NV
    ab = n // _NA

    @pl.when(k == 0)
    def _cast_w1():
        w1bf_ref[...] = w1_ref[...].astype(bf)

    us = []
    for s in range(_NV):
        u_s = jax.lax.dot_general(
            v_refs[s][...].astype(bf), w1bf_ref[...], _DN_T,
            preferred_element_type=jnp.float32).astype(bf)
        u_ref[pl.ds(k * _BK + s * vb, vb), :] = u_s
        us.append(u_s)
    u_k = jnp.concatenate(us, axis=0)

    parts = [jnp.dot(a_refs[s][...].astype(bf), u_k,
                     preferred_element_type=jnp.float32)
             for s in range(_NA)]

    @pl.when(k == 0)
    def _init():
        for s in range(_NA):
            h_ref[pl.ds(s * ab, ab), :] = parts[s]

    @pl.when(k > 0)
    def _acc():
        for s in range(_NA):
            h_ref[pl.ds(s * ab, ab), :] += parts[s]

    @pl.when(k == nb - 1)
    def _epilogue():
        h = h_ref[...] + eps_ref[0, 0] * u_ref[...].astype(jnp.float32)
        h = h + b1_ref[...]
        h = _ln_relu(h, g1_ref[...], be1_ref[...])
        h2 = jax.lax.dot_general(h.astype(bf), w2_ref[...].astype(bf), _DN_T,
                                 preferred_element_type=jnp.float32)
        h2 = h2 + b2_ref[...]
        o_ref[...] = _ln_relu(h2, g2_ref[...], be2_ref[...])


def kernel(v, a, epsilon, W1, b1, g1, be1, W2, b2, g2, be2):
    n, _ = a.shape
    hid = W1.shape[0]
    out_dim = W2.shape[0]
    nb = n // _BK
    vb = _BK // _NV
    ab = n // _NA

    row = lambda x: x.reshape(1, -1)
    const = lambda k: (0, 0)
    v_specs = [pl.BlockSpec((vb, n), functools.partial(
        lambda k, s: (_NV * k + s, 0), s=s)) for s in range(_NV)]
    a_specs = [pl.BlockSpec((ab, _BK), functools.partial(
        lambda k, s: (s, k), s=s)) for s in range(_NA)]
    out = pl.pallas_call(
        functools.partial(_gin_kernel, nb=nb, n=n),
        grid=(nb,),
        in_specs=v_specs + a_specs + [
            pl.BlockSpec((1, 1), const),
            pl.BlockSpec((hid, n), const),
            pl.BlockSpec((1, hid), const),
            pl.BlockSpec((1, hid), const),
            pl.BlockSpec((1, hid), const),
            pl.BlockSpec((out_dim, hid), const),
            pl.BlockSpec((1, out_dim), const),
            pl.BlockSpec((1, out_dim), const),
            pl.BlockSpec((1, out_dim), const),
        ],
        out_specs=pl.BlockSpec((n, out_dim), const),
        out_shape=jax.ShapeDtypeStruct((n, out_dim), jnp.float32),
        scratch_shapes=[pltpu.VMEM((n, hid), jnp.bfloat16),
                        pltpu.VMEM((n, hid), jnp.float32),
                        pltpu.VMEM((hid, n), jnp.bfloat16)],
    )(*([v] * _NV), *([a] * _NA), epsilon, W1, row(b1), row(g1), row(be1),
      W2, row(b2), row(g2), row(be2))
    return out
